# baseline (speedup 1.0000x reference)
import jax
import jax.numpy as jnp
from jax.experimental import pallas as pl
from jax.experimental.pallas import tpu as pltpu


def _vq_body(ze_ref, emb_ref, zq_ref, idx_ref, loss_ref, enorm_ref):
    b = pl.program_id(0)
    nb = pl.num_programs(0)
    emb = emb_ref[...]
    C = emb.shape[0]
    H = ze_ref.shape[2]
    W = ze_ref.shape[3]
    D = ze_ref.shape[1]
    P = H * W

    @pl.when(b == 0)
    def _():
        enorm_ref[...] = jnp.sum(emb * emb, axis=1, keepdims=True)

    ze = ze_ref[0].reshape(D, P)

    scores = jnp.dot(emb, ze, preferred_element_type=jnp.float32)
    fnorm = jnp.sum(ze * ze, axis=0, keepdims=True)
    dist = (fnorm + enorm_ref[...]) - 2.0 * scores
    idx = jnp.argmin(dist, axis=0)
    onehot = (jax.lax.broadcasted_iota(jnp.int32, (C, P), 0)
              == idx[None, :]).astype(jnp.float32)
    zq = jax.lax.dot_general(emb, onehot, (((0,), (0,)), ((), ())),
                             preferred_element_type=jnp.float32)
    zq_ref[0] = zq.reshape(D, H, W)
    for h in range(H):
        idx_ref[0, h, :] = idx[h * W:(h + 1) * W].astype(jnp.int32)

    diff = ze - zq
    part = jnp.sum(diff * diff).reshape(1, 1)

    @pl.when(b == 0)
    def _():
        loss_ref[...] = part

    @pl.when(b != 0)
    def _():
        loss_ref[...] += part

    @pl.when(b == nb - 1)
    def _():
        loss_ref[...] = loss_ref[...] / (nb * D * P)


def kernel(z_e, embed):
    B, D, H, W = z_e.shape
    C = embed.shape[0]

    zq, idx, loss = pl.pallas_call(
        _vq_body,
        grid=(B,),
        in_specs=[
            pl.BlockSpec((1, D, H, W), lambda b: (b, 0, 0, 0)),
            pl.BlockSpec((C, D), lambda b: (0, 0)),
        ],
        out_specs=[
            pl.BlockSpec((1, D, H, W), lambda b: (b, 0, 0, 0)),
            pl.BlockSpec((1, H, W), lambda b: (b, 0, 0)),
            pl.BlockSpec((1, 1), lambda b: (0, 0)),
        ],
        out_shape=[
            jax.ShapeDtypeStruct((B, D, H, W), jnp.float32),
            jax.ShapeDtypeStruct((B, H, W), jnp.int32),
            jax.ShapeDtypeStruct((1, 1), jnp.float32),
        ],
        scratch_shapes=[pltpu.VMEM((C, 1), jnp.float32)],
    )(z_e, embed)

    commitment_loss = loss.reshape(())
    return (zq, commitment_loss, idx)


# speedup vs baseline: 1.5209x; 1.5209x over previous
import jax
import jax.numpy as jnp
from jax.experimental import pallas as pl
from jax.experimental.pallas import tpu as pltpu

_BPS = 2


def _vq_body(ze_ref, emb_ref, zq_ref, idx_ref, loss_ref, enorm_ref):
    g = pl.program_id(0)
    ng = pl.num_programs(0)
    emb = emb_ref[...]
    C = emb.shape[0]
    D = ze_ref.shape[1]
    P = ze_ref.shape[2]

    @pl.when(g == 0)
    def _():
        enorm_ref[...] = jnp.sum(emb * emb, axis=1, keepdims=True)

    part = jnp.zeros((1, 1), jnp.float32)
    for i in range(_BPS):
        ze = ze_ref[i]
        scores = jnp.dot(emb, ze, preferred_element_type=jnp.float32)
        fnorm = jnp.sum(ze * ze, axis=0, keepdims=True)
        dist = (fnorm + enorm_ref[...]) - 2.0 * scores
        idx = jnp.argmin(dist, axis=0)
        onehot = (jax.lax.broadcasted_iota(jnp.int32, (C, P), 0)
                  == idx[None, :]).astype(jnp.float32)
        zq = jax.lax.dot_general(emb, onehot, (((0,), (0,)), ((), ())),
                                 preferred_element_type=jnp.float32)
        zq_ref[i] = zq
        idx_ref[i] = idx.reshape(1, P).astype(jnp.int32)
        diff = ze - zq
        part = part + jnp.sum(diff * diff).reshape(1, 1)

    @pl.when(g == 0)
    def _():
        loss_ref[...] = part

    @pl.when(g != 0)
    def _():
        loss_ref[...] += part

    @pl.when(g == ng - 1)
    def _():
        loss_ref[...] = loss_ref[...] / (ng * _BPS * D * P)


def kernel(z_e, embed):
    B, D, H, W = z_e.shape
    P = H * W
    C = embed.shape[0]
    ze = z_e.reshape(B, D, P)

    zq, idx, loss = pl.pallas_call(
        _vq_body,
        grid=(B // _BPS,),
        in_specs=[
            pl.BlockSpec((_BPS, D, P), lambda g: (g, 0, 0)),
            pl.BlockSpec((C, D), lambda g: (0, 0)),
        ],
        out_specs=[
            pl.BlockSpec((_BPS, D, P), lambda g: (g, 0, 0)),
            pl.BlockSpec((_BPS, 1, P), lambda g: (g, 0, 0)),
            pl.BlockSpec((1, 1), lambda g: (0, 0)),
        ],
        out_shape=[
            jax.ShapeDtypeStruct((B, D, P), jnp.float32),
            jax.ShapeDtypeStruct((B, 1, P), jnp.int32),
            jax.ShapeDtypeStruct((1, 1), jnp.float32),
        ],
        scratch_shapes=[pltpu.VMEM((C, 1), jnp.float32)],
    )(ze, embed)

    z_q_st = zq.reshape(B, D, H, W)
    commitment_loss = loss.reshape(())
    indices_out = idx.reshape(B, H, W)
    return (z_q_st, commitment_loss, indices_out)


# speedup vs baseline: 1.5669x; 1.0302x over previous
import jax
import jax.numpy as jnp
from jax.experimental import pallas as pl
from jax.experimental.pallas import tpu as pltpu

_BPS = 4


def _vq_body(ze_ref, emb_ref, zq_ref, idx_ref, loss_ref, enorm_ref):
    g = pl.program_id(0)
    ng = pl.num_programs(0)
    emb = emb_ref[...]
    C = emb.shape[0]
    D = ze_ref.shape[1]
    P = ze_ref.shape[2]

    @pl.when(g == 0)
    def _():
        enorm_ref[...] = jnp.sum(emb * emb, axis=1, keepdims=True)

    part = jnp.zeros((1, 1), jnp.float32)
    for i in range(_BPS):
        ze = ze_ref[i]
        scores = jnp.dot(emb, ze, preferred_element_type=jnp.float32)
        fnorm = jnp.sum(ze * ze, axis=0, keepdims=True)
        dist = (fnorm + enorm_ref[...]) - 2.0 * scores
        idx = jnp.argmin(dist, axis=0)
        onehot = (jax.lax.broadcasted_iota(jnp.int32, (C, P), 0)
                  == idx[None, :]).astype(jnp.float32)
        zq = jax.lax.dot_general(emb, onehot, (((0,), (0,)), ((), ())),
                                 preferred_element_type=jnp.float32)
        zq_ref[i] = zq
        idx_ref[i] = idx.reshape(1, P).astype(jnp.int32)
        diff = ze - zq
        part = part + jnp.sum(diff * diff).reshape(1, 1)

    @pl.when(g == 0)
    def _():
        loss_ref[...] = part

    @pl.when(g != 0)
    def _():
        loss_ref[...] += part

    @pl.when(g == ng - 1)
    def _():
        loss_ref[...] = loss_ref[...] / (ng * _BPS * D * P)


def kernel(z_e, embed):
    B, D, H, W = z_e.shape
    P = H * W
    C = embed.shape[0]
    ze = z_e.reshape(B, D, P)

    zq, idx, loss = pl.pallas_call(
        _vq_body,
        grid=(B // _BPS,),
        in_specs=[
            pl.BlockSpec((_BPS, D, P), lambda g: (g, 0, 0)),
            pl.BlockSpec((C, D), lambda g: (0, 0)),
        ],
        out_specs=[
            pl.BlockSpec((_BPS, D, P), lambda g: (g, 0, 0)),
            pl.BlockSpec((_BPS, 1, P), lambda g: (g, 0, 0)),
            pl.BlockSpec((1, 1), lambda g: (0, 0)),
        ],
        out_shape=[
            jax.ShapeDtypeStruct((B, D, P), jnp.float32),
            jax.ShapeDtypeStruct((B, 1, P), jnp.int32),
            jax.ShapeDtypeStruct((1, 1), jnp.float32),
        ],
        scratch_shapes=[pltpu.VMEM((C, 1), jnp.float32)],
    )(ze, embed)

    z_q_st = zq.reshape(B, D, H, W)
    commitment_loss = loss.reshape(())
    indices_out = idx.reshape(B, H, W)
    return (z_q_st, commitment_loss, indices_out)


# speedup vs baseline: 1.5789x; 1.0077x over previous
import jax
import jax.numpy as jnp
from jax.experimental import pallas as pl
from jax.experimental.pallas import tpu as pltpu

_BPS = 8


def _vq_body(ze_ref, emb_ref, zq_ref, idx_ref, loss_ref, enorm_ref):
    g = pl.program_id(0)
    ng = pl.num_programs(0)
    emb = emb_ref[...]
    C = emb.shape[0]
    D = ze_ref.shape[1]
    P = ze_ref.shape[2]

    @pl.when(g == 0)
    def _():
        enorm_ref[...] = jnp.sum(emb * emb, axis=1, keepdims=True)

    part = jnp.zeros((1, 1), jnp.float32)
    for i in range(_BPS):
        ze = ze_ref[i]
        scores = jnp.dot(emb, ze, preferred_element_type=jnp.float32)
        fnorm = jnp.sum(ze * ze, axis=0, keepdims=True)
        dist = (fnorm + enorm_ref[...]) - 2.0 * scores
        idx = jnp.argmin(dist, axis=0)
        onehot = (jax.lax.broadcasted_iota(jnp.int32, (C, P), 0)
                  == idx[None, :]).astype(jnp.float32)
        zq = jax.lax.dot_general(emb, onehot, (((0,), (0,)), ((), ())),
                                 preferred_element_type=jnp.float32)
        zq_ref[i] = zq
        idx_ref[i] = idx.reshape(1, P).astype(jnp.int32)
        diff = ze - zq
        part = part + jnp.sum(diff * diff).reshape(1, 1)

    @pl.when(g == 0)
    def _():
        loss_ref[...] = part

    @pl.when(g != 0)
    def _():
        loss_ref[...] += part

    @pl.when(g == ng - 1)
    def _():
        loss_ref[...] = loss_ref[...] / (ng * _BPS * D * P)


def kernel(z_e, embed):
    B, D, H, W = z_e.shape
    P = H * W
    C = embed.shape[0]
    ze = z_e.reshape(B, D, P)

    zq, idx, loss = pl.pallas_call(
        _vq_body,
        grid=(B // _BPS,),
        in_specs=[
            pl.BlockSpec((_BPS, D, P), lambda g: (g, 0, 0)),
            pl.BlockSpec((C, D), lambda g: (0, 0)),
        ],
        out_specs=[
            pl.BlockSpec((_BPS, D, P), lambda g: (g, 0, 0)),
            pl.BlockSpec((_BPS, 1, P), lambda g: (g, 0, 0)),
            pl.BlockSpec((1, 1), lambda g: (0, 0)),
        ],
        out_shape=[
            jax.ShapeDtypeStruct((B, D, P), jnp.float32),
            jax.ShapeDtypeStruct((B, 1, P), jnp.int32),
            jax.ShapeDtypeStruct((1, 1), jnp.float32),
        ],
        scratch_shapes=[pltpu.VMEM((C, 1), jnp.float32)],
    )(ze, embed)

    z_q_st = zq.reshape(B, D, H, W)
    commitment_loss = loss.reshape(())
    indices_out = idx.reshape(B, H, W)
    return (z_q_st, commitment_loss, indices_out)
